# Initial kernel scaffold; baseline (speedup 1.0000x reference)
import jax
import jax.numpy as jnp
from jax.experimental import pallas as pl


def kernel(x, Wg, Wu, Wv, Wo):
    raise NotImplementedError("write your pallas kernel here")


# speedup vs baseline: 4.7140x; 4.7140x over previous
import functools

import jax
import jax.numpy as jnp
from jax.experimental import pallas as pl
from jax.experimental.pallas import tpu as pltpu

D_MODEL = 1024
N_EXPERTS = 8
RANK = 256
TOKEN_TILE = 256


def _moe_dense_kernel(x_ref, wg_ref, wu_ref, wv_ref, wo_ref, out_ref):
    e = pl.program_id(1)
    xb = x_ref[...]

    logits = jnp.dot(xb, wg_ref[...].T, preferred_element_type=jnp.float32)
    idx = jax.lax.broadcasted_iota(jnp.int32, logits.shape, 1)
    m1 = jnp.max(logits, axis=-1, keepdims=True)
    a1 = jnp.min(jnp.where(logits == m1, idx, N_EXPERTS), axis=-1, keepdims=True)
    logits2 = jnp.where(idx == a1, -jnp.inf, logits)
    m2 = jnp.max(logits2, axis=-1, keepdims=True)
    a2 = jnp.min(jnp.where(logits2 == m2, idx, N_EXPERTS), axis=-1, keepdims=True)
    t = jnp.exp(m2 - m1)
    w1 = 1.0 / (1.0 + t)
    w2 = t / (1.0 + t)
    ce = w1 * (a1 == e) + w2 * (a2 == e)

    u = jnp.dot(xb, wu_ref[0].T, preferred_element_type=jnp.float32)
    v = jnp.dot(xb, wv_ref[0].T, preferred_element_type=jnp.float32)
    s = u * jax.nn.sigmoid(u) * v
    y = jnp.dot(s, wo_ref[0].T, preferred_element_type=jnp.float32)

    @pl.when(e == 0)
    def _():
        out_ref[...] = ce * y

    @pl.when(e != 0)
    def _():
        out_ref[...] += ce * y


def kernel(x, Wg, Wu, Wv, Wo):
    B, N, D = x.shape
    x2 = x.reshape(B * N, D)
    nt = (B * N) // TOKEN_TILE

    out = pl.pallas_call(
        _moe_dense_kernel,
        grid=(nt, N_EXPERTS),
        in_specs=[
            pl.BlockSpec((TOKEN_TILE, D), lambda i, e: (i, 0)),
            pl.BlockSpec((N_EXPERTS, D), lambda i, e: (0, 0)),
            pl.BlockSpec((1, RANK, D), lambda i, e: (e, 0, 0)),
            pl.BlockSpec((1, RANK, D), lambda i, e: (e, 0, 0)),
            pl.BlockSpec((1, D, RANK), lambda i, e: (e, 0, 0)),
        ],
        out_specs=pl.BlockSpec((TOKEN_TILE, D), lambda i, e: (i, 0)),
        out_shape=jax.ShapeDtypeStruct((B * N, D), x.dtype),
    )(x2, Wg, Wu, Wv, Wo)
    return out.reshape(B, N, D)


# speedup vs baseline: 7.4450x; 1.5794x over previous
import jax
import jax.numpy as jnp
from jax.experimental import pallas as pl

D_MODEL = 1024
N_EXPERTS = 8
RANK = 256
TOKEN_TILE = 256


def _moe_dense_kernel(x_ref, xb_ref, wg_ref, wu_ref, wv_ref, wo_ref, out_ref):
    xf = x_ref[...]
    xb = xb_ref[...]

    logits = jnp.dot(xf, wg_ref[...].T, preferred_element_type=jnp.float32)
    idx = jax.lax.broadcasted_iota(jnp.int32, logits.shape, 1)
    m1 = jnp.max(logits, axis=-1, keepdims=True)
    a1 = jnp.min(jnp.where(logits == m1, idx, N_EXPERTS), axis=-1, keepdims=True)
    logits2 = jnp.where(idx == a1, -jnp.inf, logits)
    m2 = jnp.max(logits2, axis=-1, keepdims=True)
    a2 = jnp.min(jnp.where(logits2 == m2, idx, N_EXPERTS), axis=-1, keepdims=True)
    t = jnp.exp(m2 - m1)
    w1 = 1.0 / (1.0 + t)
    w2 = t / (1.0 + t)

    acc = jnp.zeros(out_ref.shape, jnp.float32)
    for e in range(N_EXPERTS):
        u = jnp.dot(xb, wu_ref[e].T, preferred_element_type=jnp.float32)
        v = jnp.dot(xb, wv_ref[e].T, preferred_element_type=jnp.float32)
        s = u * jax.nn.sigmoid(u) * v
        y = jnp.dot(s.astype(jnp.bfloat16), wo_ref[e].T,
                    preferred_element_type=jnp.float32)
        ce = w1 * (a1 == e) + w2 * (a2 == e)
        acc = acc + ce * y
    out_ref[...] = acc


def kernel(x, Wg, Wu, Wv, Wo):
    B, N, D = x.shape
    x2 = x.reshape(B * N, D)
    xb = x2.astype(jnp.bfloat16)
    nt = (B * N) // TOKEN_TILE

    out = pl.pallas_call(
        _moe_dense_kernel,
        grid=(nt,),
        in_specs=[
            pl.BlockSpec((TOKEN_TILE, D), lambda i: (i, 0)),
            pl.BlockSpec((TOKEN_TILE, D), lambda i: (i, 0)),
            pl.BlockSpec((N_EXPERTS, D), lambda i: (0, 0)),
            pl.BlockSpec((N_EXPERTS, RANK, D), lambda i: (0, 0, 0)),
            pl.BlockSpec((N_EXPERTS, RANK, D), lambda i: (0, 0, 0)),
            pl.BlockSpec((N_EXPERTS, D, RANK), lambda i: (0, 0, 0)),
        ],
        out_specs=pl.BlockSpec((TOKEN_TILE, D), lambda i: (i, 0)),
        out_shape=jax.ShapeDtypeStruct((B * N, D), x.dtype),
    )(x2, xb, Wg, Wu.astype(jnp.bfloat16), Wv.astype(jnp.bfloat16),
      Wo.astype(jnp.bfloat16))
    return out.reshape(B, N, D)


# speedup vs baseline: 8.6105x; 1.1565x over previous
import jax
import jax.numpy as jnp
from jax.experimental import pallas as pl

D_MODEL = 1024
N_EXPERTS = 8
RANK = 256
TOKEN_TILE = 256


def _moe_dense_kernel(x_ref, xb_ref, wg_ref, wu_ref, wv_ref, wo_ref, out_ref):
    xf = x_ref[...]
    xb = xb_ref[...]

    logits = jnp.dot(xf, wg_ref[...].T, preferred_element_type=jnp.float32)
    idx = jax.lax.broadcasted_iota(jnp.int32, logits.shape, 1)
    m1 = jnp.max(logits, axis=-1, keepdims=True)
    a1 = jnp.min(jnp.where(logits == m1, idx, N_EXPERTS), axis=-1, keepdims=True)
    logits2 = jnp.where(idx == a1, -jnp.inf, logits)
    m2 = jnp.max(logits2, axis=-1, keepdims=True)
    a2 = jnp.min(jnp.where(logits2 == m2, idx, N_EXPERTS), axis=-1, keepdims=True)
    t = jnp.exp(m2 - m1)
    w1 = 1.0 / (1.0 + t)
    w2 = t / (1.0 + t)

    u = jnp.dot(xb, wu_ref[...].T, preferred_element_type=jnp.float32)
    v = jnp.dot(xb, wv_ref[...].T, preferred_element_type=jnp.float32)
    s = u * jax.nn.sigmoid(u) * v

    blocks = []
    for e in range(N_EXPERTS):
        ce = w1 * (a1 == e) + w2 * (a2 == e)
        blocks.append((ce * s[:, e * RANK:(e + 1) * RANK]).astype(jnp.bfloat16))
    s_all = jnp.concatenate(blocks, axis=1)

    out_ref[...] = jnp.dot(s_all, wo_ref[...], preferred_element_type=jnp.float32)


def kernel(x, Wg, Wu, Wv, Wo):
    B, N, D = x.shape
    x2 = x.reshape(B * N, D)
    xb = x2.astype(jnp.bfloat16)
    nt = (B * N) // TOKEN_TILE
    ER = N_EXPERTS * RANK

    wu_all = Wu.reshape(ER, D).astype(jnp.bfloat16)
    wv_all = Wv.reshape(ER, D).astype(jnp.bfloat16)
    wo_all = jnp.transpose(Wo, (0, 2, 1)).reshape(ER, D).astype(jnp.bfloat16)

    out = pl.pallas_call(
        _moe_dense_kernel,
        grid=(nt,),
        in_specs=[
            pl.BlockSpec((TOKEN_TILE, D), lambda i: (i, 0)),
            pl.BlockSpec((TOKEN_TILE, D), lambda i: (i, 0)),
            pl.BlockSpec((N_EXPERTS, D), lambda i: (0, 0)),
            pl.BlockSpec((ER, D), lambda i: (0, 0)),
            pl.BlockSpec((ER, D), lambda i: (0, 0)),
            pl.BlockSpec((ER, D), lambda i: (0, 0)),
        ],
        out_specs=pl.BlockSpec((TOKEN_TILE, D), lambda i: (i, 0)),
        out_shape=jax.ShapeDtypeStruct((B * N, D), x.dtype),
    )(x2, xb, Wg, wu_all, wv_all, wo_all)
    return out.reshape(B, N, D)


# speedup vs baseline: 8.6870x; 1.0089x over previous
import jax
import jax.numpy as jnp
from jax.experimental import pallas as pl

D_MODEL = 1024
N_EXPERTS = 8
RANK = 256
TOKEN_TILE = 512


def _moe_dense_kernel(x_ref, xb_ref, wg_ref, wu_ref, wv_ref, wo_ref, out_ref):
    xf = x_ref[...]
    xb = xb_ref[...]

    logits = jnp.dot(xf, wg_ref[...].T, preferred_element_type=jnp.float32)
    idx = jax.lax.broadcasted_iota(jnp.int32, logits.shape, 1)
    m1 = jnp.max(logits, axis=-1, keepdims=True)
    a1 = jnp.min(jnp.where(logits == m1, idx, N_EXPERTS), axis=-1, keepdims=True)
    logits2 = jnp.where(idx == a1, -jnp.inf, logits)
    m2 = jnp.max(logits2, axis=-1, keepdims=True)
    a2 = jnp.min(jnp.where(logits2 == m2, idx, N_EXPERTS), axis=-1, keepdims=True)
    t = jnp.exp(m2 - m1)
    w1 = 1.0 / (1.0 + t)
    w2 = t / (1.0 + t)

    u = jnp.dot(xb, wu_ref[...].T, preferred_element_type=jnp.float32)
    v = jnp.dot(xb, wv_ref[...].T, preferred_element_type=jnp.float32)
    s = u * jax.nn.sigmoid(u) * v

    blocks = []
    for e in range(N_EXPERTS):
        ce = w1 * (a1 == e) + w2 * (a2 == e)
        blocks.append((ce * s[:, e * RANK:(e + 1) * RANK]).astype(jnp.bfloat16))
    s_all = jnp.concatenate(blocks, axis=1)

    out_ref[...] = jnp.dot(s_all, wo_ref[...], preferred_element_type=jnp.float32)


def kernel(x, Wg, Wu, Wv, Wo):
    B, N, D = x.shape
    x2 = x.reshape(B * N, D)
    xb = x2.astype(jnp.bfloat16)
    nt = (B * N) // TOKEN_TILE
    ER = N_EXPERTS * RANK

    wu_all = Wu.reshape(ER, D).astype(jnp.bfloat16)
    wv_all = Wv.reshape(ER, D).astype(jnp.bfloat16)
    wo_all = jnp.transpose(Wo, (0, 2, 1)).reshape(ER, D).astype(jnp.bfloat16)

    out = pl.pallas_call(
        _moe_dense_kernel,
        grid=(nt,),
        in_specs=[
            pl.BlockSpec((TOKEN_TILE, D), lambda i: (i, 0)),
            pl.BlockSpec((TOKEN_TILE, D), lambda i: (i, 0)),
            pl.BlockSpec((N_EXPERTS, D), lambda i: (0, 0)),
            pl.BlockSpec((ER, D), lambda i: (0, 0)),
            pl.BlockSpec((ER, D), lambda i: (0, 0)),
            pl.BlockSpec((ER, D), lambda i: (0, 0)),
        ],
        out_specs=pl.BlockSpec((TOKEN_TILE, D), lambda i: (i, 0)),
        out_shape=jax.ShapeDtypeStruct((B * N, D), x.dtype),
    )(x2, xb, Wg, wu_all, wv_all, wo_all)
    return out.reshape(B, N, D)
